# baseline (speedup 1.0000x reference)
import functools

import jax
import jax.numpy as jnp
from jax import lax
from jax.experimental import pallas as pl
from jax.experimental.pallas import tpu as pltpu
from jax.experimental.pallas import tpu_sc as plsc

N0, N1, N2 = 50000, 10000, 2000
E1, E2 = 320000, 64000
F = 128
NC, NS = 2, 16
NW = NC * NS
K = 80
G = 25
DEPTH = 4
ZR = 128


def _pad_rows(n):
    return -(-n // (NS * ZR)) * (NS * ZR)


def _pad_edges(e):
    return -(-e // (NW * G * K)) * (NW * G * K)


@functools.lru_cache(maxsize=None)
def _make_segsum(E_pad, N_out):
    CH = E_pad // (NW * K)
    GR = CH // G
    N_pad = _pad_rows(N_out)
    persub = N_pad // NS
    mesh = plsc.VectorSubcoreMesh(
        core_axis_name="c", subcore_axis_name="s",
        num_cores=NC, num_subcores=NS)

    @functools.partial(
        pl.kernel,
        out_type=jax.ShapeDtypeStruct((NC * N_pad, F), jnp.float32),
        mesh=mesh,
        scratch_types=[
            pltpu.VMEM((G, K), jnp.int32),
            pltpu.VMEM((G, K), jnp.int32),
            pltpu.VMEM((DEPTH * K, F), jnp.float32),
            pltpu.VMEM_SHARED((N_pad, F), jnp.float32),
            pltpu.SemaphoreType.DMA((DEPTH,)),
        ],
    )
    def segsum(x_hbm, edges_hbm, zeros_hbm, out_hbm,
               src_v, dst_v, rows, acc_sh, sg):
        cid = lax.axis_index("c")
        sid = lax.axis_index("s")
        wid = sid * NC + cid

        pltpu.sync_copy(zeros_hbm, acc_sh.at[pl.ds(sid * persub, persub)])
        plsc.subcore_barrier()

        def group(g, _):
            pltpu.sync_copy(edges_hbm.at[0, wid, g], src_v)
            pltpu.sync_copy(edges_hbm.at[1, wid, g], dst_v)
            for p in range(DEPTH - 1):
                pltpu.async_copy(x_hbm.at[src_v.at[p]],
                                 rows.at[pl.ds(p * K, K)], sg.at[p])
            def body(c, _):
                b = c % DEPTH
                nb = (c + DEPTH - 1) % DEPTH
                @pl.when(c + DEPTH - 1 < G)
                def _():
                    pltpu.async_copy(x_hbm.at[src_v.at[c + DEPTH - 1]],
                                     rows.at[pl.ds(nb * K, K)], sg.at[nb])
                pltpu.make_async_copy(x_hbm.at[src_v.at[c]],
                                      rows.at[pl.ds(b * K, K)],
                                      sg.at[b]).wait()
                pltpu.sync_copy(rows.at[pl.ds(b * K, K)],
                                acc_sh.at[dst_v.at[c]], add=True)
                return 0
            lax.fori_loop(0, G, body, 0)
            return 0
        lax.fori_loop(0, GR, group, 0)
        plsc.subcore_barrier()

        for r in range(persub // ZR):
            off = sid * persub + r * ZR
            pltpu.sync_copy(acc_sh.at[pl.ds(off, ZR)],
                            out_hbm.at[pl.ds(cid * N_pad + off, ZR)])

    return segsum


def _segsum(x_rows, edge_index, E, N_out):
    E_pad = _pad_edges(E)
    if E_pad != E:
        pad = E_pad - E
        edge_index = jnp.concatenate(
            [edge_index,
             jnp.stack([jnp.zeros((pad,), jnp.int32),
                        jnp.full((pad,), N_out, jnp.int32)])], axis=1)
    edges = edge_index.reshape(2, NW, E_pad // (NW * G * K), G, K)
    N_pad = _pad_rows(N_out)
    zeros = jnp.zeros((N_pad // NS, F), jnp.float32)
    out = _make_segsum(E_pad, N_out)(x_rows, edges, zeros)
    return out.reshape(NC, N_pad, F)


def _root1_body(xt_ref, wroot_ref, b_ref, o_ref):
    o_ref[...] = jnp.dot(xt_ref[...], wroot_ref[...],
                         preferred_element_type=jnp.float32) + b_ref[...]


def _lin1_body(p_ref, root_ref, wr_ref, o_ref):
    a = p_ref[0] + p_ref[1]
    acc = jnp.dot(a, wr_ref[...], preferred_element_type=jnp.float32)
    o_ref[...] = jnp.maximum(acc + root_ref[...], 0.0)


def _head_body(p_ref, ht_ref, wr_ref, b2_ref, wroot_ref, wlin_ref, blin_ref,
               whead_ref, bhead_ref, o_ref):
    a = p_ref[0] + p_ref[1]
    h2 = jnp.dot(a, wr_ref[...], preferred_element_type=jnp.float32)
    h2 += jnp.dot(ht_ref[...], wroot_ref[...],
                  preferred_element_type=jnp.float32)
    h2 += b2_ref[...]
    t = jnp.dot(h2, wlin_ref[...], preferred_element_type=jnp.float32)
    t += blin_ref[...]
    o_ref[...] = jnp.dot(t, whead_ref[...],
                         preferred_element_type=jnp.float32) + bhead_ref[...]


def kernel(x, edge_index1, edge_index2, W_rel1, b_rel1, W_root1,
           W_rel2, b_rel2, W_root2, W_lin, b_lin, W_head, b_head):
    R = 1000
    root1 = pl.pallas_call(
        _root1_body,
        grid=(N1 // R,),
        in_specs=[
            pl.BlockSpec((R, F), lambda i: (i, 0)),
            pl.BlockSpec((F, F), lambda i: (0, 0)),
            pl.BlockSpec((1, F), lambda i: (0, 0)),
        ],
        out_specs=pl.BlockSpec((R, F), lambda i: (i, 0)),
        out_shape=jax.ShapeDtypeStruct((N1, F), jnp.float32),
    )(x, W_root1, b_rel1.reshape(1, F))

    p1 = _segsum(x, edge_index1, E1, N1)

    h = pl.pallas_call(
        _lin1_body,
        grid=(N1 // R,),
        in_specs=[
            pl.BlockSpec((NC, R, F), lambda i: (0, i, 0)),
            pl.BlockSpec((R, F), lambda i: (i, 0)),
            pl.BlockSpec((F, F), lambda i: (0, 0)),
        ],
        out_specs=pl.BlockSpec((R, F), lambda i: (i, 0)),
        out_shape=jax.ShapeDtypeStruct((N1, F), jnp.float32),
    )(p1, root1, W_rel1)

    p2 = _segsum(h, edge_index2, E2, N2)

    C = W_head.shape[1]
    W_head_p = jnp.zeros((F, F), jnp.float32).at[:, :C].set(W_head)
    b_head_p = jnp.zeros((1, F), jnp.float32).at[0, :C].set(b_head)
    R2 = 200
    wspec = lambda *s: pl.BlockSpec(s, lambda i: tuple(0 for _ in s))
    out = pl.pallas_call(
        _head_body,
        grid=(N2 // R2,),
        in_specs=[
            pl.BlockSpec((NC, R2, F), lambda i: (0, i, 0)),
            pl.BlockSpec((R2, F), lambda i: (i, 0)),
            wspec(F, F), wspec(1, F), wspec(F, F),
            wspec(F, F), wspec(1, F), wspec(F, F), wspec(1, F),
        ],
        out_specs=pl.BlockSpec((R2, F), lambda i: (i, 0)),
        out_shape=jax.ShapeDtypeStruct((N2, F), jnp.float32),
    )(p2, h, W_rel2, b_rel2.reshape(1, F), W_root2,
      W_lin, b_lin.reshape(1, F), W_head_p, b_head_p)
    return out[:, :C]


# speedup vs baseline: 1.0557x; 1.0557x over previous
import functools

import jax
import jax.numpy as jnp
from jax import lax
from jax.experimental import pallas as pl
from jax.experimental.pallas import tpu as pltpu
from jax.experimental.pallas import tpu_sc as plsc

N0, N1, N2 = 50000, 10000, 2000
E1, E2 = 320000, 64000
F = 128
NC, NS = 2, 16
NW = NC * NS
K = 80
G = 25
DEPTH = 3
ZR = 128


def _pad_rows(n):
    return -(-n // (NS * ZR)) * (NS * ZR)


def _pad_edges(e):
    return -(-e // (NW * G * K)) * (NW * G * K)


@functools.lru_cache(maxsize=None)
def _make_segsum(E_pad, N_out):
    CH = E_pad // (NW * K)
    GR = CH // G
    N_pad = _pad_rows(N_out)
    persub = N_pad // NS
    mesh = plsc.VectorSubcoreMesh(
        core_axis_name="c", subcore_axis_name="s",
        num_cores=NC, num_subcores=NS)

    @functools.partial(
        pl.kernel,
        out_type=jax.ShapeDtypeStruct((NC * N_pad, F), jnp.float32),
        mesh=mesh,
        scratch_types=[
            pltpu.VMEM((2, G, K), jnp.int32),
            pltpu.VMEM((2, G, K), jnp.int32),
            pltpu.VMEM((DEPTH * K, F), jnp.float32),
            pltpu.VMEM_SHARED((N_pad, F), jnp.float32),
            pltpu.SemaphoreType.DMA((DEPTH,)),
            pltpu.SemaphoreType.DMA((2,)),
        ],
    )
    def segsum(x_hbm, edges_hbm, zeros_hbm, out_hbm,
               src_v, dst_v, rows, acc_sh, sg, si):
        cid = lax.axis_index("c")
        sid = lax.axis_index("s")
        wid = sid * NC + cid

        pltpu.sync_copy(zeros_hbm, acc_sh.at[pl.ds(sid * persub, persub)])
        plsc.subcore_barrier()

        def stage(g, s):
            pltpu.async_copy(edges_hbm.at[0, wid, g], src_v.at[s], si.at[s])
            pltpu.async_copy(edges_hbm.at[1, wid, g], dst_v.at[s], si.at[s])

        def stage_wait(g, s):
            pltpu.make_async_copy(edges_hbm.at[0, wid, g], src_v.at[s],
                                  si.at[s]).wait()
            pltpu.make_async_copy(edges_hbm.at[1, wid, g], dst_v.at[s],
                                  si.at[s]).wait()

        stage(0, 0)

        def group(g, _):
            s = g % 2
            @pl.when(g + 1 < GR)
            def _():
                stage(g + 1, 1 - s)
            stage_wait(g, s)
            for p in range(DEPTH - 1):
                pltpu.async_copy(x_hbm.at[src_v.at[s, p]],
                                 rows.at[pl.ds(p * K, K)], sg.at[p])
            def body(c, _):
                b = c % DEPTH
                nb = (c + DEPTH - 1) % DEPTH
                @pl.when(c + DEPTH - 1 < G)
                def _():
                    pltpu.async_copy(x_hbm.at[src_v.at[s, c + DEPTH - 1]],
                                     rows.at[pl.ds(nb * K, K)], sg.at[nb])
                pltpu.make_async_copy(x_hbm.at[src_v.at[s, c]],
                                      rows.at[pl.ds(b * K, K)],
                                      sg.at[b]).wait()
                pltpu.sync_copy(rows.at[pl.ds(b * K, K)],
                                acc_sh.at[dst_v.at[s, c]], add=True)
                return 0
            lax.fori_loop(0, G, body, 0)
            return 0
        lax.fori_loop(0, GR, group, 0)
        plsc.subcore_barrier()

        for r in range(persub // ZR):
            off = sid * persub + r * ZR
            pltpu.sync_copy(acc_sh.at[pl.ds(off, ZR)],
                            out_hbm.at[pl.ds(cid * N_pad + off, ZR)])

    return segsum


def _segsum(x_rows, edge_index, E, N_out):
    E_pad = _pad_edges(E)
    if E_pad != E:
        pad = E_pad - E
        edge_index = jnp.concatenate(
            [edge_index,
             jnp.stack([jnp.zeros((pad,), jnp.int32),
                        jnp.full((pad,), N_out, jnp.int32)])], axis=1)
    edges = edge_index.reshape(2, NW, E_pad // (NW * G * K), G, K)
    N_pad = _pad_rows(N_out)
    zeros = jnp.zeros((N_pad // NS, F), jnp.float32)
    out = _make_segsum(E_pad, N_out)(x_rows, edges, zeros)
    return out.reshape(NC, N_pad, F)


def _root1_body(xt_ref, wroot_ref, b_ref, o_ref):
    o_ref[...] = jnp.dot(xt_ref[...], wroot_ref[...],
                         preferred_element_type=jnp.float32) + b_ref[...]


def _lin1_body(p_ref, root_ref, wr_ref, o_ref):
    a = p_ref[0] + p_ref[1]
    acc = jnp.dot(a, wr_ref[...], preferred_element_type=jnp.float32)
    o_ref[...] = jnp.maximum(acc + root_ref[...], 0.0)


def _head_body(p_ref, ht_ref, wr_ref, b2_ref, wroot_ref, wlin_ref, blin_ref,
               whead_ref, bhead_ref, o_ref):
    a = p_ref[0] + p_ref[1]
    h2 = jnp.dot(a, wr_ref[...], preferred_element_type=jnp.float32)
    h2 += jnp.dot(ht_ref[...], wroot_ref[...],
                  preferred_element_type=jnp.float32)
    h2 += b2_ref[...]
    t = jnp.dot(h2, wlin_ref[...], preferred_element_type=jnp.float32)
    t += blin_ref[...]
    o_ref[...] = jnp.dot(t, whead_ref[...],
                         preferred_element_type=jnp.float32) + bhead_ref[...]


def kernel(x, edge_index1, edge_index2, W_rel1, b_rel1, W_root1,
           W_rel2, b_rel2, W_root2, W_lin, b_lin, W_head, b_head):
    R = 1000
    root1 = pl.pallas_call(
        _root1_body,
        grid=(N1 // R,),
        in_specs=[
            pl.BlockSpec((R, F), lambda i: (i, 0)),
            pl.BlockSpec((F, F), lambda i: (0, 0)),
            pl.BlockSpec((1, F), lambda i: (0, 0)),
        ],
        out_specs=pl.BlockSpec((R, F), lambda i: (i, 0)),
        out_shape=jax.ShapeDtypeStruct((N1, F), jnp.float32),
    )(x, W_root1, b_rel1.reshape(1, F))

    p1 = _segsum(x, edge_index1, E1, N1)

    h = pl.pallas_call(
        _lin1_body,
        grid=(N1 // R,),
        in_specs=[
            pl.BlockSpec((NC, R, F), lambda i: (0, i, 0)),
            pl.BlockSpec((R, F), lambda i: (i, 0)),
            pl.BlockSpec((F, F), lambda i: (0, 0)),
        ],
        out_specs=pl.BlockSpec((R, F), lambda i: (i, 0)),
        out_shape=jax.ShapeDtypeStruct((N1, F), jnp.float32),
    )(p1, root1, W_rel1)

    p2 = _segsum(h, edge_index2, E2, N2)

    C = W_head.shape[1]
    W_head_p = jnp.zeros((F, F), jnp.float32).at[:, :C].set(W_head)
    b_head_p = jnp.zeros((1, F), jnp.float32).at[0, :C].set(b_head)
    full = lambda *s: pl.BlockSpec(s, lambda i: tuple(0 for _ in s))
    out = pl.pallas_call(
        _head_body,
        grid=(1,),
        in_specs=[
            full(NC, N2, F), full(N2, F), full(F, F), full(1, F),
            full(F, F), full(F, F), full(1, F), full(F, F), full(1, F),
        ],
        out_specs=full(N2, F),
        out_shape=jax.ShapeDtypeStruct((N2, F), jnp.float32),
    )(p2, h, W_rel2, b_rel2.reshape(1, F), W_root2,
      W_lin, b_lin.reshape(1, F), W_head_p, b_head_p)
    return out[:, :C]
